# Initial kernel scaffold; baseline (speedup 1.0000x reference)
import jax
import jax.numpy as jnp
from jax.experimental import pallas as pl


def kernel(x, edge_index, W1, b1, W2, b2, W3, b3, W4, b4, Wfc, bfc):
    raise NotImplementedError("write your pallas kernel here")


# speedup vs baseline: 4.3997x; 4.3997x over previous
import dataclasses
import functools

import jax
import jax.numpy as jnp
from jax import lax
from jax.experimental import pallas as pl
from jax.experimental.pallas import tpu as pltpu
from jax.experimental.pallas import tpu_sc as plsc

NC = 2
NS = 16
NW = NC * NS
CH = 80
NP = 10240
_BN = 1280


def _sc_mesh():
    return plsc.VectorSubcoreMesh(core_axis_name="c", subcore_axis_name="s")


def _sc_compiler_params():
    cp = pltpu.CompilerParams()
    if "needs_layout_passes" in pltpu.CompilerParams.__dataclass_fields__:
        cp = dataclasses.replace(cp, needs_layout_passes=False)
    return cp


def _degrees_sc(src, dst, zeros1d):
    e = src.shape[0]
    epw = e // NW
    cpw = epw // CH

    @functools.partial(
        pl.kernel,
        out_type=[
            jax.ShapeDtypeStruct((NW, NP), jnp.float32),
            jax.ShapeDtypeStruct((NW, NP), jnp.float32),
        ],
        mesh=_sc_mesh(),
        compiler_params=_sc_compiler_params(),
        scratch_types=[
            pltpu.VMEM((CH,), jnp.int32),
            pltpu.VMEM((CH,), jnp.int32),
            pltpu.VMEM((NP,), jnp.float32),
            pltpu.VMEM((NP,), jnp.float32),
        ],
    )
    def deg_kernel(src_hbm, dst_hbm, z_hbm, out_s_hbm, out_d_hbm,
                   idx_s, idx_d, tab_s, tab_d):
        cid = lax.axis_index("c")
        sid = lax.axis_index("s")
        pltpu.sync_copy(z_hbm, tab_s)
        pltpu.sync_copy(z_hbm, tab_d)
        wid = cid * NS + sid
        base = wid * epw
        ones_vec = jnp.full((16,), 1.0, jnp.float32)

        @pl.loop(0, cpw)
        def _(c):
            off = base + c * CH
            pltpu.sync_copy(src_hbm.at[pl.ds(off, CH)], idx_s)
            pltpu.sync_copy(dst_hbm.at[pl.ds(off, CH)], idx_d)
            for k in range(CH // 16):
                iv_s = idx_s[pl.ds(k * 16, 16)]
                iv_d = idx_d[pl.ds(k * 16, 16)]
                plsc.addupdate_scatter(tab_s, [iv_s], ones_vec)
                plsc.addupdate_scatter(tab_d, [iv_d], ones_vec)

        pltpu.sync_copy(tab_s, out_s_hbm.at[wid])
        pltpu.sync_copy(tab_d, out_d_hbm.at[wid])

    return deg_kernel(src, dst, zeros1d)


def _agg_sc(v, src, dst, zeros):
    d = v.shape[1]
    e = src.shape[0]
    epw = e // NW
    cpw = epw // CH
    rps = NP // NS

    @functools.partial(
        pl.kernel,
        out_type=jax.ShapeDtypeStruct((NC, NP, d), jnp.float32),
        mesh=_sc_mesh(),
        scratch_types=[
            pltpu.VMEM((CH,), jnp.int32),
            pltpu.VMEM((CH,), jnp.int32),
            pltpu.VMEM((CH, d), jnp.float32),
            pltpu.VMEM_SHARED((NP, d), jnp.float32),
        ],
    )
    def agg_kernel(v_hbm, src_hbm, dst_hbm, z_hbm, out_hbm, idx_s, idx_d,
                   rows, tab):
        cid = lax.axis_index("c")
        sid = lax.axis_index("s")
        pltpu.sync_copy(z_hbm.at[pl.ds(sid * rps, rps)],
                        tab.at[pl.ds(sid * rps, rps)])
        plsc.subcore_barrier()
        base = (cid * NS + sid) * epw

        @pl.loop(0, cpw)
        def _(c):
            off = base + c * CH
            pltpu.sync_copy(src_hbm.at[pl.ds(off, CH)], idx_s)
            pltpu.sync_copy(dst_hbm.at[pl.ds(off, CH)], idx_d)
            pltpu.sync_copy(v_hbm.at[idx_s], rows)
            pltpu.sync_copy(rows, tab.at[idx_d], add=True)

        plsc.subcore_barrier()
        pltpu.sync_copy(tab.at[pl.ds(sid * rps, rps)],
                        out_hbm.at[cid, pl.ds(sid * rps, rps)])

    return agg_kernel(v, src, dst, zeros)


def _prep_tc(od_parts, id_parts, xp):
    d = xp.shape[1]

    def body(od_ref, id_ref, x_ref, ns_ref, nd_ref, v_ref):
        od = jnp.sum(od_ref[...], axis=0)
        idg = jnp.sum(id_ref[...], axis=0)
        ns = lax.rsqrt(jnp.maximum(od, 1.0))
        nd = lax.rsqrt(jnp.maximum(idg, 1.0))
        ns_ref[...] = jnp.broadcast_to(ns[:, None], (_BN, 16))
        nd_ref[...] = jnp.broadcast_to(nd[:, None], (_BN, 16))
        v_ref[...] = x_ref[...] * ns[:, None]

    return pl.pallas_call(
        body,
        grid=(NP // _BN,),
        in_specs=[
            pl.BlockSpec((NW, _BN), lambda i: (0, i)),
            pl.BlockSpec((NW, _BN), lambda i: (0, i)),
            pl.BlockSpec((_BN, d), lambda i: (i, 0)),
        ],
        out_specs=[
            pl.BlockSpec((_BN, 16), lambda i: (i, 0)),
            pl.BlockSpec((_BN, 16), lambda i: (i, 0)),
            pl.BlockSpec((_BN, d), lambda i: (i, 0)),
        ],
        out_shape=[
            jax.ShapeDtypeStruct((NP, 16), jnp.float32),
            jax.ShapeDtypeStruct((NP, 16), jnp.float32),
            jax.ShapeDtypeStruct((NP, d), jnp.float32),
        ],
    )(od_parts, id_parts, xp)


def _layer_tc(t_parts, nd, ns, xp, W, b):
    d = xp.shape[1]

    def body(t_ref, nd_ref, ns_ref, x_ref, w_ref, b_ref, o_ref):
        t = (t_ref[0] + t_ref[1]) * nd_ref[:, :1]
        h = lax.dot_general(t, w_ref[...], (((1,), (0,)), ((), ())),
                            precision=lax.Precision.HIGHEST)
        h = jnp.maximum(h + b_ref[...], 0.0) + x_ref[...]
        o_ref[...] = h * ns_ref[:, :1]

    return pl.pallas_call(
        body,
        grid=(NP // _BN,),
        in_specs=[
            pl.BlockSpec((NC, _BN, d), lambda i: (0, i, 0)),
            pl.BlockSpec((_BN, 16), lambda i: (i, 0)),
            pl.BlockSpec((_BN, 16), lambda i: (i, 0)),
            pl.BlockSpec((_BN, d), lambda i: (i, 0)),
            pl.BlockSpec((d, d), lambda i: (0, 0)),
            pl.BlockSpec((1, d), lambda i: (0, 0)),
        ],
        out_specs=pl.BlockSpec((_BN, d), lambda i: (i, 0)),
        out_shape=jax.ShapeDtypeStruct((NP, d), jnp.float32),
    )(t_parts, nd, ns, xp, W, b)


def _final_tc(t_parts, nd, W4, b4, Wfc, bfc):
    d = W4.shape[0]
    c = Wfc.shape[1]

    def body(t_ref, nd_ref, w4_ref, b4_ref, wfc_ref, bfc_ref, o_ref):
        t = (t_ref[0] + t_ref[1]) * nd_ref[:, :1]
        h = lax.dot_general(t, w4_ref[...], (((1,), (0,)), ((), ())),
                            precision=lax.Precision.HIGHEST)
        h = jnp.maximum(h + b4_ref[...], 0.0)
        o_ref[...] = lax.dot_general(h, wfc_ref[...], (((1,), (0,)), ((), ())),
                                     precision=lax.Precision.HIGHEST) + bfc_ref[...]

    return pl.pallas_call(
        body,
        grid=(NP // _BN,),
        in_specs=[
            pl.BlockSpec((NC, _BN, d), lambda i: (0, i, 0)),
            pl.BlockSpec((_BN, 16), lambda i: (i, 0)),
            pl.BlockSpec((d, d), lambda i: (0, 0)),
            pl.BlockSpec((1, d), lambda i: (0, 0)),
            pl.BlockSpec((d, c), lambda i: (0, 0)),
            pl.BlockSpec((1, c), lambda i: (0, 0)),
        ],
        out_specs=pl.BlockSpec((_BN, c), lambda i: (i, 0)),
        out_shape=jax.ShapeDtypeStruct((NP, c), jnp.float32),
    )(t_parts, nd, W4, b4, Wfc, bfc)


def kernel(x, edge_index, W1, b1, W2, b2, W3, b3, W4, b4, Wfc, bfc):
    n, d = x.shape
    e = edge_index.shape[1]
    c = Wfc.shape[1]
    src = edge_index[0]
    dst = edge_index[1]
    zeros = jnp.zeros((NP, d), jnp.float32)
    zeros1d = jnp.zeros((NP,), jnp.float32)
    xp = jnp.pad(x, ((0, NP - n), (0, 0)))

    od_parts, id_parts = _degrees_sc(src, dst, zeros1d)
    ns, nd, v = _prep_tc(od_parts, id_parts, xp)
    for Wk, bk in ((W1, b1), (W2, b2), (W3, b3)):
        t_parts = _agg_sc(v, src, dst, zeros)
        v = _layer_tc(t_parts, nd, ns, xp, Wk, bk.reshape(1, d))
    t_parts = _agg_sc(v, src, dst, zeros)
    y = _final_tc(t_parts, nd, W4, b4.reshape(1, d), Wfc, bfc.reshape(1, c))
    return y[:n]
